# baseline (speedup 1.0000x reference)
import functools

import jax
import jax.numpy as jnp
from jax import lax
from jax.experimental import pallas as pl
from jax.experimental.pallas import tpu as pltpu
from jax.experimental.pallas import tpu_sc as plsc

_N_OUT = 4194304
_N_IDX = 1048576
_N_TILES = 16
_IDX_PER_TILE = _N_IDX // _N_TILES
_CB = 2048
_NCH = _IDX_PER_TILE // _CB
_R0 = 1896320
_R1 = (_N_OUT - 2 * _R0) // 2
_TRASH = 4096
_ZB = 4096


def _zchunks(words):
    out, off = [], 0
    while off < words:
        sz = min(_ZB, words - off)
        out.append((off, sz))
        off += sz
    return out


def _dchunks(words):
    out, off = [], 0
    while off < words:
        sz = min(_CB, words - off)
        out.append((off, sz))
        off += sz
    return out


def _sc_scatter(time_idxs):
    mesh = plsc.VectorSubcoreMesh(core_axis_name="c", subcore_axis_name="s")

    @functools.partial(
        pl.kernel,
        out_type=jax.ShapeDtypeStruct((_N_OUT,), jnp.int32),
        mesh=mesh,
        scratch_types=[
            pltpu.VMEM_SHARED((_R0 + _TRASH,), jnp.int32),
            [pltpu.VMEM((_CB,), jnp.int32) for _ in range(3)],
            pltpu.VMEM((_CB,), jnp.int32),
            pltpu.VMEM((_ZB,), jnp.int32),
            pltpu.SemaphoreType.DMA,
            pltpu.SemaphoreType.DMA,
            pltpu.SemaphoreType.DMA,
            pltpu.SemaphoreType.DMA,
            pltpu.SemaphoreType.DMA,
        ],
    )
    def k(idx_hbm, out_hbm, mask_sh, idx_vs, ones_v, zeros_v,
          sem_i, sem_z, sem_s, sem_d1, sem_d2):
        c = lax.axis_index("c")
        t = lax.axis_index("s")
        ibase = t * _IDX_PER_TILE

        def idx_cp(q, buf):
            return pltpu.make_async_copy(
                idx_hbm.at[pl.ds(ibase + q * _CB, _CB)], idx_vs[buf], sem_i)

        ones16 = jnp.ones((16,), jnp.int32)
        zeros16 = jnp.zeros((16,), jnp.int32)

        def const_init(i, carry):
            for u in range(2):
                off = (i * 2 + u) * 16
                ones_v[pl.ds(off, 16)] = ones16
                zeros_v[pl.ds(off, 16)] = zeros16
                zeros_v[pl.ds(_CB + off, 16)] = zeros16
            return carry

        lax.fori_loop(0, _CB // 32, const_init, 0)

        for p in range(2):
            if p == 0:
                size = _R0
                lo = c * _R0
            else:
                size = _R1
                lo = 2 * _R0 + c * _R1
            ztile = size // _N_TILES

            idx_cp(0, 0).start()
            idx_cp(1, 1).start()

            zcps = []
            for off, sz in _zchunks(ztile):
                cp = pltpu.make_async_copy(
                    zeros_v.at[pl.ds(0, sz)],
                    mask_sh.at[pl.ds(t * ztile + off, sz)], sem_z)
                cp.start()
                zcps.append(cp)
            for cp in zcps:
                cp.wait()

            plsc.subcore_barrier()

            scps = [None] * _NCH
            waited = [False] * _NCH
            for q in range(_NCH):
                b = q % 3
                idx_cp(q, b).wait()

                def remap(i, carry, _b=b, _lo=lo, _size=size):
                    for u in range(4):
                        off = (i * 4 + u) * 16
                        v = idx_vs[_b][pl.ds(off, 16)]
                        m = (v >= _lo) & (v < _lo + _size)
                        tgt = jnp.where(m, v - _lo, _R0 + (v & (_TRASH - 1)))
                        idx_vs[_b][pl.ds(off, 16)] = tgt
                    return carry

                lax.fori_loop(0, _CB // 64, remap, 0)

                cp = pltpu.make_async_copy(
                    ones_v, mask_sh.at[idx_vs[b]], sem_s)
                cp.start()
                scps[q] = cp
                if q + 2 < _NCH:
                    if q >= 1:
                        scps[q - 1].wait()
                        waited[q - 1] = True
                    idx_cp(q + 2, (q + 2) % 3).start()
            for q in range(_NCH):
                if not waited[q]:
                    scps[q].wait()

            plsc.subcore_barrier()

            chunks = _dchunks(ztile)
            h1s = [None] * len(chunks)
            h2s = [None] * len(chunks)

            def h1(kk, buf):
                off, sz = chunks[kk]
                return pltpu.make_async_copy(
                    mask_sh.at[pl.ds(t * ztile + off, sz)],
                    idx_vs[buf].at[pl.ds(0, sz)], sem_d1)

            def h2(kk, buf):
                off, sz = chunks[kk]
                return pltpu.make_async_copy(
                    idx_vs[buf].at[pl.ds(0, sz)],
                    out_hbm.at[pl.ds(lo + t * ztile + off, sz)], sem_d2)

            h1s[0] = h1(0, 0)
            h1s[0].start()
            if len(chunks) >= 2:
                h1s[1] = h1(1, 1)
                h1s[1].start()
            for kk in range(len(chunks)):
                b = kk % 3
                h1s[kk].wait()
                cp = h2(kk, b)
                cp.start()
                h2s[kk] = cp
                if kk + 2 < len(chunks):
                    if kk >= 1:
                        h2s[kk - 1].wait()
                    nxt = h1(kk + 2, (kk + 2) % 3)
                    nxt.start()
                    h1s[kk + 2] = nxt
            nch = len(chunks)
            for kk in range(max(0, nch - 3), nch):
                h2s[kk].wait()

            plsc.subcore_barrier()

    return k(time_idxs)


def kernel(time_idxs, n):
    out = _sc_scatter(time_idxs)
    return (out != 0) & (jnp.asarray(n) > 0)


# speedup vs baseline: 1.0394x; 1.0394x over previous
import functools

import jax
import jax.numpy as jnp
from jax import lax
from jax.experimental import pallas as pl
from jax.experimental.pallas import tpu as pltpu
from jax.experimental.pallas import tpu_sc as plsc

_N_OUT = 4194304
_N_IDX = 1048576
_N_TILES = 16
_IDX_PER_TILE = _N_IDX // _N_TILES
_CB = 4096
_NCH = _IDX_PER_TILE // _CB
_R0 = 1768832
_R1 = (_N_OUT - 2 * _R0) // 2
_TRASH = 512
_ZB = 4096


def _zchunks(words):
    out, off = [], 0
    while off < words:
        sz = min(_ZB, words - off)
        out.append((off, sz))
        off += sz
    return out


def _dchunks(words):
    out, off = [], 0
    while off < words:
        sz = min(_CB, words - off)
        out.append((off, sz))
        off += sz
    return out


def _sc_scatter(time_idxs):
    mesh = plsc.VectorSubcoreMesh(core_axis_name="c", subcore_axis_name="s")

    @functools.partial(
        pl.kernel,
        out_type=jax.ShapeDtypeStruct((_N_OUT,), jnp.int32),
        mesh=mesh,
        scratch_types=[
            pltpu.VMEM_SHARED((_R0 + _TRASH,), jnp.int32),
            [pltpu.VMEM((_CB,), jnp.int32) for _ in range(3)],
            pltpu.VMEM((_CB,), jnp.int32),
            pltpu.VMEM((_ZB,), jnp.int32),
            pltpu.SemaphoreType.DMA,
            pltpu.SemaphoreType.DMA,
            pltpu.SemaphoreType.DMA,
            pltpu.SemaphoreType.DMA,
            pltpu.SemaphoreType.DMA,
        ],
    )
    def k(idx_hbm, out_hbm, mask_sh, idx_vs, ones_v, zeros_v,
          sem_i, sem_z, sem_s, sem_d1, sem_d2):
        c = lax.axis_index("c")
        t = lax.axis_index("s")
        ibase = t * _IDX_PER_TILE

        def idx_cp(q, buf):
            return pltpu.make_async_copy(
                idx_hbm.at[pl.ds(ibase + q * _CB, _CB)], idx_vs[buf], sem_i)

        ones16 = jnp.ones((16,), jnp.int32)
        zeros16 = jnp.zeros((16,), jnp.int32)

        def const_init(i, carry):
            for u in range(2):
                off = (i * 2 + u) * 16
                ones_v[pl.ds(off, 16)] = ones16
                zeros_v[pl.ds(off, 16)] = zeros16
            return carry

        lax.fori_loop(0, _CB // 32, const_init, 0)
        if _ZB > _CB:
            def zext_init(i, carry):
                for u in range(2):
                    off = _CB + (i * 2 + u) * 16
                    zeros_v[pl.ds(off, 16)] = zeros16
                return carry

            lax.fori_loop(0, (_ZB - _CB) // 32, zext_init, 0)

        for p in range(2):
            if p == 0:
                size = _R0
                lo = c * _R0
            else:
                size = _R1
                lo = 2 * _R0 + c * _R1
            ztile = size // _N_TILES

            idx_cp(0, 0).start()
            idx_cp(1, 1).start()

            zcps = []
            for off, sz in _zchunks(ztile):
                cp = pltpu.make_async_copy(
                    zeros_v.at[pl.ds(0, sz)],
                    mask_sh.at[pl.ds(t * ztile + off, sz)], sem_z)
                cp.start()
                zcps.append(cp)
            for cp in zcps:
                cp.wait()

            plsc.subcore_barrier()

            scps = [None] * _NCH
            waited = [False] * _NCH
            for q in range(_NCH):
                b = q % 3
                idx_cp(q, b).wait()

                def remap(i, carry, _b=b, _lo=lo, _size=size):
                    for u in range(4):
                        off = (i * 4 + u) * 16
                        v = idx_vs[_b][pl.ds(off, 16)]
                        m = (v >= _lo) & (v < _lo + _size)
                        tgt = jnp.where(m, v - _lo, _R0 + (v & (_TRASH - 1)))
                        idx_vs[_b][pl.ds(off, 16)] = tgt
                    return carry

                lax.fori_loop(0, _CB // 64, remap, 0)

                cp = pltpu.make_async_copy(
                    ones_v, mask_sh.at[idx_vs[b]], sem_s)
                cp.start()
                scps[q] = cp
                if q + 2 < _NCH:
                    if q >= 1:
                        scps[q - 1].wait()
                        waited[q - 1] = True
                    idx_cp(q + 2, (q + 2) % 3).start()
            for q in range(_NCH):
                if not waited[q]:
                    scps[q].wait()

            plsc.subcore_barrier()

            chunks = _dchunks(ztile)
            h1s = [None] * len(chunks)
            h2s = [None] * len(chunks)

            def h1(kk, buf):
                off, sz = chunks[kk]
                return pltpu.make_async_copy(
                    mask_sh.at[pl.ds(t * ztile + off, sz)],
                    idx_vs[buf].at[pl.ds(0, sz)], sem_d1)

            def h2(kk, buf):
                off, sz = chunks[kk]
                return pltpu.make_async_copy(
                    idx_vs[buf].at[pl.ds(0, sz)],
                    out_hbm.at[pl.ds(lo + t * ztile + off, sz)], sem_d2)

            h1s[0] = h1(0, 0)
            h1s[0].start()
            if len(chunks) >= 2:
                h1s[1] = h1(1, 1)
                h1s[1].start()
            for kk in range(len(chunks)):
                b = kk % 3
                h1s[kk].wait()
                cp = h2(kk, b)
                cp.start()
                h2s[kk] = cp
                if kk + 2 < len(chunks):
                    if kk >= 1:
                        h2s[kk - 1].wait()
                    nxt = h1(kk + 2, (kk + 2) % 3)
                    nxt.start()
                    h1s[kk + 2] = nxt
            nch = len(chunks)
            for kk in range(max(0, nch - 3), nch):
                h2s[kk].wait()

            plsc.subcore_barrier()

    return k(time_idxs)


def kernel(time_idxs, n):
    out = _sc_scatter(time_idxs)
    return (out != 0) & (jnp.asarray(n) > 0)
